# baseline (speedup 1.0000x reference)
import functools

import jax
import jax.numpy as jnp
from jax import lax
from jax.experimental import pallas as pl
from jax.experimental.pallas import tpu as pltpu
from jax.experimental.pallas import tpu_sc as plsc

_B, _T, _D = 4, 1024, 32
_N = _B * _T
_K = 1024
_BLK = 1024
_GRID = _N // _BLK

_SC_INFO = plsc.get_sparse_core_info()
_NC = _SC_INFO.num_cores
_NS = _SC_INFO.num_subcores
_NW = _NC * _NS
_BPW = _N // _NW
_L = 16


def _vq_body(z_ref, cb_ref, idx_ref, cbt_ref):
    @pl.when(pl.program_id(0) == 0)
    def _():
        cbt_ref[...] = cb_ref[...].T
    zb = z_ref[...]
    cbt = cbt_ref[...]
    cnorm = jnp.sum(cbt * cbt, axis=0)
    dots = lax.dot_general(
        zb, cbt, (((1,), (0,)), ((), ())),
        precision=lax.Precision.HIGHEST,
        preferred_element_type=jnp.float32)
    scores = cnorm[None, :] - 2.0 * dots
    idx_ref[0, 0, :] = jnp.argmin(scores, axis=1).astype(jnp.int32)


_SC_MESH = plsc.VectorSubcoreMesh(core_axis_name="c", subcore_axis_name="s")


@functools.partial(
    pl.kernel,
    mesh=_SC_MESH,
    out_type=jax.ShapeDtypeStruct((_B, _T, _D), jnp.float32),
    scratch_types=[
        pltpu.VMEM((_BPW,), jnp.int32),
        pltpu.VMEM((_BPW,), jnp.int32),
        pltpu.VMEM((_BPW, 128), jnp.float32),
        pltpu.VMEM((_BPW, _D), jnp.float32),
        pltpu.SemaphoreType.DMA,
    ],
    compiler_params=pltpu.CompilerParams(needs_layout_passes=False),
)
def _sc_gather(cb4_hbm, idx_hbm, out_hbm, idx_v, q_v, rows_v, zq_v, sem):
    wid = lax.axis_index("s") * _NC + lax.axis_index("c")
    base = wid * _BPW
    b = base // _T
    t0 = base % _T
    pltpu.sync_copy(idx_hbm.at[b, 0, pl.ds(t0, _BPW)], idx_v)
    for i in range(_BPW // _L):
        q_v[pl.ds(i * _L, _L)] = lax.shift_right_logical(
            idx_v[pl.ds(i * _L, _L)], 2)
    pltpu.async_copy(cb4_hbm.at[q_v], rows_v, sem).wait()
    for i in range(_BPW // _L):
        tok = lax.iota(jnp.int32, _L) + i * _L
        off = (idx_v[pl.ds(i * _L, _L)] & 3) * _D
        for j in range(_D):
            vals = plsc.load_gather(rows_v, [tok, off + j])
            plsc.store_scatter(zq_v, [tok, lax.iota(jnp.int32, _L) * 0 + j],
                               vals)
    pltpu.sync_copy(zq_v, out_hbm.at[b, pl.ds(t0, _BPW), :])


@jax.jit
def kernel(z, codebook):
    zf = z.reshape(_N, _D)
    idx3 = pl.pallas_call(
        _vq_body,
        grid=(_GRID,),
        in_specs=[
            pl.BlockSpec((_BLK, _D), lambda i: (i, 0)),
            pl.BlockSpec((_K, _D), lambda i: (0, 0)),
        ],
        out_specs=pl.BlockSpec((1, 1, _BLK), lambda i: (i, 0, 0)),
        out_shape=jax.ShapeDtypeStruct((_GRID, 1, _BLK), jnp.int32),
        scratch_shapes=[pltpu.VMEM((_D, _K), jnp.float32)],
    )(zf, codebook)
    zq = _sc_gather(codebook.reshape(256, 128), idx3)
    return zq, idx3.reshape(_B, _T)


# speedup vs baseline: 1.3130x; 1.3130x over previous
import functools

import jax
import jax.numpy as jnp
from jax import lax
from jax.experimental import pallas as pl
from jax.experimental.pallas import tpu as pltpu
from jax.experimental.pallas import tpu_sc as plsc

_B, _T, _D = 4, 1024, 32
_N = _B * _T
_K = 1024
_BLK = 1024
_GRID = _N // _BLK

_SC_INFO = plsc.get_sparse_core_info()
_NC = _SC_INFO.num_cores
_NS = _SC_INFO.num_subcores
_NW = _NC * _NS
_BPW = _N // _NW


def _vq_body(z_ref, cb_ref, idx_ref, cbt_ref):
    @pl.when(pl.program_id(0) == 0)
    def _():
        cbt_ref[...] = cb_ref[...].T
    zb = z_ref[0]
    cbt = cbt_ref[...]
    cnorm = jnp.sum(cbt * cbt, axis=0)
    dots = lax.dot_general(
        zb, cbt, (((1,), (0,)), ((), ())),
        precision=lax.Precision.HIGHEST,
        preferred_element_type=jnp.float32)
    scores = cnorm[None, :] - 2.0 * dots
    idx_ref[0, 0, :] = jnp.argmin(scores, axis=1).astype(jnp.int32)


_SC_MESH = plsc.VectorSubcoreMesh(core_axis_name="c", subcore_axis_name="s")


@functools.partial(
    pl.kernel,
    mesh=_SC_MESH,
    out_type=jax.ShapeDtypeStruct((_N, _D), jnp.float32),
    scratch_types=[
        pltpu.VMEM((_BPW,), jnp.int32),
        pltpu.VMEM((_BPW, _D), jnp.float32),
        pltpu.SemaphoreType.DMA,
    ],
    compiler_params=pltpu.CompilerParams(use_tc_tiling_on_sc=False),
)
def _sc_gather(cb_hbm, idx_hbm, out_hbm, idx_v, rows_v, sem):
    wid = lax.axis_index("s") * _NC + lax.axis_index("c")
    base = wid * _BPW
    pltpu.sync_copy(idx_hbm.at[pl.ds(base, _BPW)], idx_v)
    pltpu.async_copy(cb_hbm.at[idx_v], rows_v, sem).wait()
    pltpu.sync_copy(rows_v, out_hbm.at[pl.ds(base, _BPW)])


@jax.jit
def kernel(z, codebook):
    idx3 = pl.pallas_call(
        _vq_body,
        grid=(_GRID,),
        in_specs=[
            pl.BlockSpec((1, _BLK, _D), lambda i: (i, 0, 0)),
            pl.BlockSpec((_K, _D), lambda i: (0, 0)),
        ],
        out_specs=pl.BlockSpec((1, 1, _BLK), lambda i: (i, 0, 0)),
        out_shape=jax.ShapeDtypeStruct((_GRID, 1, _BLK), jnp.int32),
        scratch_shapes=[pltpu.VMEM((_D, _K), jnp.float32)],
    )(z, codebook)
    idx = idx3.reshape(_N)
    zq = _sc_gather(codebook, idx)
    return zq.reshape(_B, _T, _D), idx3.reshape(_B, _T)


# speedup vs baseline: 1.3167x; 1.0028x over previous
import functools

import jax
import jax.numpy as jnp
from jax import lax
from jax.experimental import pallas as pl
from jax.experimental.pallas import tpu as pltpu
from jax.experimental.pallas import tpu_sc as plsc

_B, _T, _D = 4, 1024, 32
_N = _B * _T
_K = 1024
_BLK = 1024
_GRID = _N // _BLK

_SC_INFO = plsc.get_sparse_core_info()
_NC = _SC_INFO.num_cores
_NS = _SC_INFO.num_subcores
_NW = _NC * _NS
_BPW = _N // _NW


def _vq_body(z_ref, cb_ref, idx_ref, cbt_ref):
    @pl.when(pl.program_id(0) == 0)
    def _():
        cbt_ref[...] = cb_ref[...].T
    zb = z_ref[0]
    cbt = cbt_ref[...]
    cnorm = jnp.sum(cbt * cbt, axis=0)
    dots = lax.dot_general(
        zb, cbt, (((1,), (0,)), ((), ())),
        precision=lax.Precision.HIGHEST,
        preferred_element_type=jnp.float32)
    scores = cnorm[None, :] - 2.0 * dots
    idx_ref[0, 0, :] = jnp.argmin(scores, axis=1).astype(jnp.int32)


_SC_MESH = plsc.VectorSubcoreMesh(core_axis_name="c", subcore_axis_name="s")


@functools.partial(
    pl.kernel,
    mesh=_SC_MESH,
    out_type=jax.ShapeDtypeStruct((_B, _T, _D), jnp.float32),
    scratch_types=[
        pltpu.VMEM((_BPW,), jnp.int32),
        pltpu.VMEM((_BPW, _D), jnp.float32),
        pltpu.SemaphoreType.DMA,
    ],
    compiler_params=pltpu.CompilerParams(use_tc_tiling_on_sc=False),
)
def _sc_gather(cb_hbm, idx_hbm, out_hbm, idx_v, rows_v, sem):
    wid = lax.axis_index("s") * _NC + lax.axis_index("c")
    base = wid * _BPW
    b = base // _T
    t0 = base % _T
    pltpu.sync_copy(idx_hbm.at[pl.ds(base, _BPW)], idx_v)
    pltpu.async_copy(cb_hbm.at[idx_v], rows_v, sem).wait()
    pltpu.sync_copy(rows_v, out_hbm.at[b, pl.ds(t0, _BPW), :])


@jax.jit
def kernel(z, codebook):
    idx3 = pl.pallas_call(
        _vq_body,
        grid=(_GRID,),
        in_specs=[
            pl.BlockSpec((1, _BLK, _D), lambda i: (i, 0, 0)),
            pl.BlockSpec((_K, _D), lambda i: (0, 0)),
        ],
        out_specs=pl.BlockSpec((1, 1, _BLK), lambda i: (i, 0, 0)),
        out_shape=jax.ShapeDtypeStruct((_GRID, 1, _BLK), jnp.int32),
        scratch_shapes=[pltpu.VMEM((_D, _K), jnp.float32)],
    )(z, codebook)
    idx = idx3.reshape(_N)
    zq = _sc_gather(codebook, idx)
    return zq, idx3.reshape(_B, _T)


# speedup vs baseline: 1.3878x; 1.0540x over previous
import functools

import jax
import jax.numpy as jnp
from jax import lax
from jax.experimental import pallas as pl
from jax.experimental.pallas import tpu as pltpu
from jax.experimental.pallas import tpu_sc as plsc

_B, _T, _D = 4, 1024, 32
_N = _B * _T
_K = 1024
_BLK = 1024
_GRID = _N // _BLK

_SC_INFO = plsc.get_sparse_core_info()
_NC = _SC_INFO.num_cores
_NS = _SC_INFO.num_subcores
_NW = _NC * _NS
_BPW = _N // _NW


def _vq_body(zt_ref, cbt_ref, idx_ref):
    zbt = zt_ref[0]
    cbt = cbt_ref[...]
    cnorm = jnp.sum(cbt * cbt, axis=0)
    dots = lax.dot_general(
        zbt, cbt, (((0,), (0,)), ((), ())),
        precision=lax.Precision.HIGHEST,
        preferred_element_type=jnp.float32)
    scores = cnorm[None, :] - 2.0 * dots
    idx_ref[0, 0, :] = jnp.argmin(scores, axis=1).astype(jnp.int32)


_SC_MESH = plsc.VectorSubcoreMesh(core_axis_name="c", subcore_axis_name="s")


@functools.partial(
    pl.kernel,
    mesh=_SC_MESH,
    out_type=jax.ShapeDtypeStruct((_B, _T, _D), jnp.float32),
    scratch_types=[
        pltpu.VMEM((_BPW,), jnp.int32),
        pltpu.VMEM((_BPW, _D), jnp.float32),
        pltpu.SemaphoreType.DMA,
    ],
    compiler_params=pltpu.CompilerParams(use_tc_tiling_on_sc=False),
)
def _sc_gather(cb_hbm, idx_hbm, out_hbm, idx_v, rows_v, sem):
    wid = lax.axis_index("s") * _NC + lax.axis_index("c")
    base = wid * _BPW
    b = base // _T
    t0 = base % _T
    pltpu.sync_copy(idx_hbm.at[pl.ds(base, _BPW)], idx_v)
    pltpu.async_copy(cb_hbm.at[idx_v], rows_v, sem).wait()
    pltpu.sync_copy(rows_v, out_hbm.at[b, pl.ds(t0, _BPW), :])


@jax.jit
def kernel(z, codebook):
    idx3 = pl.pallas_call(
        _vq_body,
        grid=(_GRID,),
        in_specs=[
            pl.BlockSpec((1, _D, _BLK), lambda i: (i, 0, 0)),
            pl.BlockSpec((_D, _K), lambda i: (0, 0)),
        ],
        out_specs=pl.BlockSpec((1, 1, _BLK), lambda i: (i, 0, 0)),
        out_shape=jax.ShapeDtypeStruct((_GRID, 1, _BLK), jnp.int32),
    )(z.transpose(0, 2, 1), codebook.T)
    idx = idx3.reshape(_N)
    zq = _sc_gather(codebook, idx)
    return zq, idx3.reshape(_B, _T)


# speedup vs baseline: 1.3879x; 1.0000x over previous
import functools

import jax
import jax.numpy as jnp
from jax import lax
from jax.experimental import pallas as pl
from jax.experimental.pallas import tpu as pltpu
from jax.experimental.pallas import tpu_sc as plsc

_B, _T, _D = 4, 1024, 32
_N = _B * _T
_K = 1024
_BLK = 1024
_GRID = _N // _BLK

_SC_INFO = plsc.get_sparse_core_info()
_NC = _SC_INFO.num_cores
_NS = _SC_INFO.num_subcores
_NW = _NC * _NS
_BPW = _N // _NW
_HPW = _BPW // 2


def _vq_body(zt_ref, cbt_ref, idx_ref):
    zbt = zt_ref[0]
    cbt = cbt_ref[...]
    cnorm = jnp.sum(cbt * cbt, axis=0)
    dots = lax.dot_general(
        zbt, cbt, (((0,), (0,)), ((), ())),
        precision=lax.Precision.HIGHEST,
        preferred_element_type=jnp.float32)
    scores = cnorm[None, :] - 2.0 * dots
    idx_ref[0, 0, :] = jnp.argmin(scores, axis=1).astype(jnp.int32)


_SC_MESH = plsc.VectorSubcoreMesh(core_axis_name="c", subcore_axis_name="s")


@functools.partial(
    pl.kernel,
    mesh=_SC_MESH,
    out_type=jax.ShapeDtypeStruct((_B, _T, _D), jnp.float32),
    scratch_types=[
        pltpu.VMEM((_HPW,), jnp.int32),
        pltpu.VMEM((_HPW,), jnp.int32),
        pltpu.VMEM((_HPW, _D), jnp.float32),
        pltpu.VMEM((_HPW, _D), jnp.float32),
        pltpu.SemaphoreType.DMA,
        pltpu.SemaphoreType.DMA,
    ],
    compiler_params=pltpu.CompilerParams(use_tc_tiling_on_sc=False),
)
def _sc_gather(cb_hbm, idx_hbm, out_hbm, idx0_v, idx1_v, rows0_v, rows1_v,
               sem0, sem1):
    wid = lax.axis_index("s") * _NC + lax.axis_index("c")
    base = wid * _BPW
    b = base // _T
    t0 = base % _T
    i0 = pltpu.async_copy(idx_hbm.at[pl.ds(base, _HPW)], idx0_v, sem0)
    i1 = pltpu.async_copy(idx_hbm.at[pl.ds(base + _HPW, _HPW)], idx1_v, sem1)
    i0.wait()
    g0 = pltpu.async_copy(cb_hbm.at[idx0_v], rows0_v, sem0)
    i1.wait()
    g1 = pltpu.async_copy(cb_hbm.at[idx1_v], rows1_v, sem1)
    g0.wait()
    o0 = pltpu.async_copy(rows0_v, out_hbm.at[b, pl.ds(t0, _HPW), :], sem0)
    g1.wait()
    o1 = pltpu.async_copy(rows1_v, out_hbm.at[b, pl.ds(t0 + _HPW, _HPW), :],
                          sem1)
    o0.wait()
    o1.wait()


@jax.jit
def kernel(z, codebook):
    idx3 = pl.pallas_call(
        _vq_body,
        grid=(_GRID,),
        in_specs=[
            pl.BlockSpec((1, _D, _BLK), lambda i: (i, 0, 0)),
            pl.BlockSpec((_D, _K), lambda i: (0, 0)),
        ],
        out_specs=pl.BlockSpec((1, 1, _BLK), lambda i: (i, 0, 0)),
        out_shape=jax.ShapeDtypeStruct((_GRID, 1, _BLK), jnp.int32),
    )(z.transpose(0, 2, 1), codebook.T)
    idx = idx3.reshape(_N)
    zq = _sc_gather(codebook, idx)
    return zq, idx3.reshape(_B, _T)
